# baseline (speedup 1.0000x reference)
import math
from functools import partial

import jax
import jax.numpy as jnp
from jax import lax
from jax.experimental import pallas as pl
from jax.experimental.pallas import tpu as pltpu

EPS = 1e-5
GIMG = 8


def _shift_lanes(x, s, zcol):
    if s == 0:
        return x
    if s > 0:
        return jnp.concatenate([x[:, s:], zcol[:, :s]], axis=1)
    return jnp.concatenate([zcol[:, :(-s)], x[:, :s]], axis=1)


def _p1_kernel(x_ref, w_ref, b3_ref, y1_ref, st_ref, *, G, W, Cin, Cout):
    x4 = x_ref[...]
    hw = x4.shape[2] * x4.shape[3]
    xb = x4.reshape(G * Cin, hw).astype(jnp.bfloat16)
    rows = G * Cin
    zcol = jnp.zeros((rows, 33), jnp.bfloat16)
    lane = lax.broadcasted_iota(jnp.int32, (1, hw), 1) % W
    zero = jnp.zeros((), jnp.bfloat16)
    shifted = []
    for ky in range(3):
        for kx in range(3):
            s = W * (ky - 1) + (kx - 1)
            t = _shift_lanes(xb, s, zcol)
            if kx == 0:
                t = jnp.where(lane == 0, zero, t)
            elif kx == 2:
                t = jnp.where(lane == W - 1, zero, t)
            shifted.append(t)
    b3c = b3_ref[:, 0:1]
    for i in range(G):
        r0 = i * Cin
        acc = jnp.dot(w_ref[0], shifted[0][r0:r0 + Cin, :],
                      preferred_element_type=jnp.float32)
        for k in range(1, 9):
            acc = acc + jnp.dot(w_ref[k], shifted[k][r0:r0 + Cin, :],
                                preferred_element_type=jnp.float32)
        y = acc + b3c
        y1_ref[0, i * Cout:(i + 1) * Cout, :] = y
        s1 = jnp.sum(y, axis=1, keepdims=True)
        s2 = jnp.sum(y * y, axis=1, keepdims=True)
        if i == 0:
            st1, st2 = s1, s2
        else:
            st1, st2 = st1 + s1, st2 + s2
    st_ref[0] = jnp.concatenate([st1, st2], axis=1)


def _p2_kernel(x_ref, y1_ref, w1_ref, ss_ref, o_ref, *, G, Cin, Cout, H, W):
    x4 = x_ref[...]
    hw = H * W
    xb = x4.reshape(G * Cin, hw).astype(jnp.bfloat16)
    ss = ss_ref[...]
    sc = ss[0][:, 0:1]
    sh = ss[1][:, 0:1]
    b1c = ss[2][:, 0:1]
    res = []
    for i in range(G):
        y2 = jnp.dot(w1_ref[...], xb[i * Cin:(i + 1) * Cin, :],
                     preferred_element_type=jnp.float32)
        y1 = y1_ref[0, i * Cout:(i + 1) * Cout, :]
        y1n = jnp.maximum(y1 * sc + sh, 0.0)
        res.append(jnp.maximum(y1n + y2 + b1c, 0.0))
    o_ref[...] = jnp.concatenate(res, axis=0).reshape(G, Cout, H, W)


@jax.jit
def _forward(x_nchw, w3, b3, gamma, beta, w1, b1):
    N, Cin, H, W = x_nchw.shape
    Cout = w3.shape[-1]
    HW = H * W
    P = N * HW
    g = math.gcd(GIMG, N)
    ng = N // g

    x = x_nchw.astype(jnp.float32)
    w9 = jnp.transpose(w3.astype(jnp.float32),
                       (0, 1, 3, 2)).reshape(9, Cout, Cin).astype(jnp.bfloat16)
    w1t = jnp.transpose(w1.astype(jnp.float32)).astype(jnp.bfloat16)
    b3b = jnp.broadcast_to(b3.reshape(Cout, 1).astype(jnp.float32),
                           (Cout, 128))

    cparams = pltpu.CompilerParams(
        dimension_semantics=("parallel",),
        vmem_limit_bytes=64 * 1024 * 1024,
    )

    flops1 = int(N * 9 * Cout * Cin * HW * 2 + N * 6 * Cout * HW)
    bytes1 = int(4 * (N * Cin * HW + N * Cout * HW) + 2 * 9 * Cout * Cin
                 + 4 * (Cout * 128 + ng * Cout * 2))
    y1, stats = pl.pallas_call(
        partial(_p1_kernel, G=g, W=W, Cin=Cin, Cout=Cout),
        grid=(ng,),
        in_specs=[
            pl.BlockSpec((g, Cin, H, W), lambda n: (n, 0, 0, 0)),
            pl.BlockSpec((9, Cout, Cin), lambda n: (0, 0, 0)),
            pl.BlockSpec((Cout, 128), lambda n: (0, 0)),
        ],
        out_specs=(
            pl.BlockSpec((1, g * Cout, HW), lambda n: (n, 0, 0)),
            pl.BlockSpec((1, Cout, 2), lambda n: (n, 0, 0)),
        ),
        out_shape=(
            jax.ShapeDtypeStruct((ng, g * Cout, HW), jnp.float32),
            jax.ShapeDtypeStruct((ng, Cout, 2), jnp.float32),
        ),
        compiler_params=cparams,
        cost_estimate=pl.CostEstimate(flops=flops1, transcendentals=0,
                                      bytes_accessed=bytes1),
    )(x, w9, b3b)

    s = stats.sum(axis=0)
    mean = s[:, 0] / P
    var = s[:, 1] / P - mean * mean
    scale = gamma.reshape(Cout) * lax.rsqrt(var + EPS)
    shift = beta.reshape(Cout) - mean * scale
    ssb = jnp.broadcast_to(
        jnp.stack([scale, shift, b1.reshape(Cout).astype(jnp.float32)]
                  )[:, :, None], (3, Cout, 128))

    flops2 = int(N * Cout * Cin * HW * 2 + N * 6 * Cout * HW)
    bytes2 = int(4 * (N * Cin * HW + 2 * N * Cout * HW) + 2 * Cout * Cin
                 + 4 * 3 * Cout * 128)
    out = pl.pallas_call(
        partial(_p2_kernel, G=g, Cin=Cin, Cout=Cout, H=H, W=W),
        grid=(ng,),
        in_specs=[
            pl.BlockSpec((g, Cin, H, W), lambda n: (n, 0, 0, 0)),
            pl.BlockSpec((1, g * Cout, HW), lambda n: (n, 0, 0)),
            pl.BlockSpec((Cout, Cin), lambda n: (0, 0)),
            pl.BlockSpec((3, Cout, 128), lambda n: (0, 0, 0)),
        ],
        out_specs=pl.BlockSpec((g, Cout, H, W), lambda n: (n, 0, 0, 0)),
        out_shape=jax.ShapeDtypeStruct((N, Cout, H, W), jnp.float32),
        compiler_params=cparams,
        cost_estimate=pl.CostEstimate(flops=flops2, transcendentals=0,
                                      bytes_accessed=bytes2),
    )(x, y1, w1t, ssb)

    return out


def kernel(x_nchw, w3, b3, gamma, beta, w1, b1):
    return _forward(x_nchw, w3, b3, gamma, beta, w1, b1)


# speedup vs baseline: 1.1511x; 1.1511x over previous
import math
from functools import partial

import jax
import jax.numpy as jnp
from jax import lax
from jax.experimental import pallas as pl
from jax.experimental.pallas import tpu as pltpu

EPS = 1e-5
WT = 4
GIMG = 8


def _band3_tile(w3):
    cin, cout = w3.shape[2], w3.shape[3]
    pi = jnp.arange(WT + 2)[:, None]
    po = jnp.arange(WT)[None, :]
    kx = pi - po
    valid = ((kx >= 0) & (kx < 3)).astype(w3.dtype)
    g = w3[:, jnp.clip(kx, 0, 2)]
    g = g * valid[None, :, :, None, None]
    g = jnp.transpose(g, (0, 1, 3, 2, 4))
    return g.reshape(3, (WT + 2) * cin, WT * cout)


def _band1_tile(w1):
    cin, cout = w1.shape
    pi = jnp.arange(WT + 2)[:, None]
    po = jnp.arange(WT)[None, :]
    sel = (pi == po + 1).astype(w1.dtype)
    g = sel[:, None, :, None] * w1[None, :, None, :]
    return g.reshape((WT + 2) * cin, WT * cout)


def _tile_lanes(v, w):
    return jnp.tile(v.reshape(1, -1), (1, w))


def _p1_kernel(x_ref, w_ref, b3_ref, y1_ref, st_ref, *, H, W, Cin, Cout):
    x = x_ref[0]
    gh = x.shape[0]
    zpad = jnp.zeros((gh, Cin), x.dtype)
    xp = jnp.concatenate([zpad, x, zpad], axis=1)
    zrow = jnp.zeros((1, xp.shape[1]), x.dtype)
    row = lax.broadcasted_iota(jnp.int32, (gh, 1), 0) % H
    zero = jnp.zeros((), x.dtype)
    xup = jnp.where(row == 0, zero, jnp.concatenate([zrow, xp[:gh - 1]], 0))
    xdn = jnp.where(row == H - 1, zero, jnp.concatenate([xp[1:], zrow], 0))
    kt = (WT + 2) * Cin
    ot = WT * Cout
    sums, sqs = [], []
    for t in range(W // WT):
        si = t * WT * Cin
        so = t * WT * Cout
        a = (jnp.dot(xup[:, si:si + kt], w_ref[0],
                     preferred_element_type=jnp.float32)
             + jnp.dot(xp[:, si:si + kt], w_ref[1],
                       preferred_element_type=jnp.float32)
             + jnp.dot(xdn[:, si:si + kt], w_ref[2],
                       preferred_element_type=jnp.float32))
        y = a + b3_ref[:, so:so + ot]
        y1_ref[0, :, so:so + ot] = y.astype(jnp.bfloat16)
        sums.append(jnp.sum(y, axis=0, keepdims=True))
        sqs.append(jnp.sum(y * y, axis=0, keepdims=True))
    st_ref[0] = jnp.concatenate(
        [jnp.concatenate(sums, axis=1), jnp.concatenate(sqs, axis=1)], axis=0)


def _p2_kernel(x_ref, y1_ref, w1_ref, b1_ref, sc_ref, sh_ref, o_ref,
               *, W, Cin, Cout):
    x = x_ref[0]
    gh = x.shape[0]
    zpad = jnp.zeros((gh, Cin), x.dtype)
    xp = jnp.concatenate([zpad, x, zpad], axis=1)
    kt = (WT + 2) * Cin
    ot = WT * Cout
    for t in range(W // WT):
        si = t * WT * Cin
        so = t * WT * Cout
        y2 = jnp.dot(xp[:, si:si + kt], w1_ref[...],
                     preferred_element_type=jnp.float32)
        y1 = y1_ref[0, :, so:so + ot].astype(jnp.float32)
        y1n = jnp.maximum(
            y1 * sc_ref[:, so:so + ot] + sh_ref[:, so:so + ot], 0.0)
        o_ref[0, :, so:so + ot] = jnp.maximum(
            y1n + y2 + b1_ref[:, so:so + ot], 0.0)


@jax.jit
def _forward(x_nchw, w3, b3, gamma, beta, w1, b1):
    N, Cin, H, W = x_nchw.shape
    Cout = w3.shape[-1]
    WCin, WCout = W * Cin, W * Cout
    P = N * H * W
    g = math.gcd(GIMG, N)
    ng = N // g
    gh = g * H

    x = jnp.transpose(x_nchw, (0, 2, 3, 1)).reshape(ng, gh, WCin)
    x = x.astype(jnp.bfloat16)
    w3b = _band3_tile(w3.astype(jnp.float32)).astype(jnp.bfloat16)
    w1b = _band1_tile(w1.astype(jnp.float32)).astype(jnp.bfloat16)
    b3t = _tile_lanes(b3, W).astype(jnp.float32)
    b1t = _tile_lanes(b1, W).astype(jnp.float32)

    cparams = pltpu.CompilerParams(
        dimension_semantics=("parallel",),
        vmem_limit_bytes=64 * 1024 * 1024,
    )

    kt = (WT + 2) * Cin
    ot = WT * Cout
    nt = W // WT

    flops1 = int(ng * nt * 3 * gh * kt * ot * 2 + N * 6 * H * WCout)
    bytes1 = int(2 * N * H * WCin + 2 * 3 * kt * ot
                 + 4 * WCout + 2 * N * H * WCout + 4 * ng * 2 * WCout)
    y1, stats = pl.pallas_call(
        partial(_p1_kernel, H=H, W=W, Cin=Cin, Cout=Cout),
        grid=(ng,),
        in_specs=[
            pl.BlockSpec((1, gh, WCin), lambda n: (n, 0, 0)),
            pl.BlockSpec((3, kt, ot), lambda n: (0, 0, 0)),
            pl.BlockSpec((1, WCout), lambda n: (0, 0)),
        ],
        out_specs=(
            pl.BlockSpec((1, gh, WCout), lambda n: (n, 0, 0)),
            pl.BlockSpec((1, 2, WCout), lambda n: (n, 0, 0)),
        ),
        out_shape=(
            jax.ShapeDtypeStruct((ng, gh, WCout), jnp.bfloat16),
            jax.ShapeDtypeStruct((ng, 2, WCout), jnp.float32),
        ),
        compiler_params=cparams,
        cost_estimate=pl.CostEstimate(flops=flops1, transcendentals=0,
                                      bytes_accessed=bytes1),
    )(x, w3b, b3t)

    s = stats.sum(axis=0).reshape(2, W, Cout).sum(axis=1)
    mean = s[0] / P
    var = s[1] / P - mean * mean
    scale = gamma.reshape(Cout) * lax.rsqrt(var + EPS)
    shift = beta.reshape(Cout) - mean * scale
    sc = _tile_lanes(scale, W).astype(jnp.float32)
    sh = _tile_lanes(shift, W).astype(jnp.float32)

    flops2 = int(ng * nt * gh * kt * ot * 2 + N * 6 * H * WCout)
    bytes2 = int(2 * N * H * WCin + 2 * N * H * WCout + 2 * kt * ot
                 + 4 * 3 * WCout + 4 * N * H * WCout)
    out = pl.pallas_call(
        partial(_p2_kernel, W=W, Cin=Cin, Cout=Cout),
        grid=(ng,),
        in_specs=[
            pl.BlockSpec((1, gh, WCin), lambda n: (n, 0, 0)),
            pl.BlockSpec((1, gh, WCout), lambda n: (n, 0, 0)),
            pl.BlockSpec((kt, ot), lambda n: (0, 0)),
            pl.BlockSpec((1, WCout), lambda n: (0, 0)),
            pl.BlockSpec((1, WCout), lambda n: (0, 0)),
            pl.BlockSpec((1, WCout), lambda n: (0, 0)),
        ],
        out_specs=pl.BlockSpec((1, gh, WCout), lambda n: (n, 0, 0)),
        out_shape=jax.ShapeDtypeStruct((ng, gh, WCout), jnp.float32),
        compiler_params=cparams,
        cost_estimate=pl.CostEstimate(flops=flops2, transcendentals=0,
                                      bytes_accessed=bytes2),
    )(x, y1, w1b, b1t, sc, sh)

    out = out.reshape(N, H, W, Cout)
    return jnp.transpose(out, (0, 3, 1, 2))


def kernel(x_nchw, w3, b3, gamma, beta, w1, b1):
    return _forward(x_nchw, w3, b3, gamma, beta, w1, b1)


# speedup vs baseline: 2.6791x; 2.3275x over previous
import math
from functools import partial

import jax
import jax.numpy as jnp
from jax import lax
from jax.experimental import pallas as pl
from jax.experimental.pallas import tpu as pltpu

EPS = 1e-5
GIMG = 8
PB = 128


def _shift_lanes(x, s, zcol):
    if s == 0:
        return x
    if s > 0:
        return jnp.concatenate([x[:, s:], zcol[:, :s]], axis=1)
    return jnp.concatenate([zcol[:, :(-s)], x[:, :s]], axis=1)


def _tin_kernel(x_ref, o_ref):
    o_ref[...] = jnp.transpose(x_ref[...], (2, 0, 1)).astype(jnp.bfloat16)


def _tout_kernel(x_ref, o_ref):
    o_ref[...] = jnp.transpose(x_ref[...], (1, 2, 0)).astype(jnp.float32)


def _p1_kernel(x_ref, w_ref, b3_ref, y1_ref, st_ref, *, G, W, Cin, Cout):
    xb = x_ref[0]
    rows, hw = xb.shape
    zcol = jnp.zeros((rows, 33), jnp.bfloat16)
    lane = lax.broadcasted_iota(jnp.int32, (1, hw), 1) % W
    zero = jnp.zeros((), jnp.bfloat16)
    shifted = []
    for ky in range(3):
        for kx in range(3):
            s = W * (ky - 1) + (kx - 1)
            t = _shift_lanes(xb, s, zcol)
            if kx == 0:
                t = jnp.where(lane == 0, zero, t)
            elif kx == 2:
                t = jnp.where(lane == W - 1, zero, t)
            shifted.append(t)
    b3c = b3_ref[:, 0:1]
    for i in range(G):
        r0 = i * Cin
        acc = jnp.dot(w_ref[0], shifted[0][r0:r0 + Cin, :],
                      preferred_element_type=jnp.float32)
        for k in range(1, 9):
            acc = acc + jnp.dot(w_ref[k], shifted[k][r0:r0 + Cin, :],
                                preferred_element_type=jnp.float32)
        y = acc + b3c
        y1_ref[0, i * Cout:(i + 1) * Cout, :] = y.astype(jnp.bfloat16)
        s1 = jnp.sum(y, axis=1, keepdims=True)
        s2 = jnp.sum(y * y, axis=1, keepdims=True)
        if i == 0:
            st1, st2 = s1, s2
        else:
            st1, st2 = st1 + s1, st2 + s2
    st_ref[0] = jnp.concatenate([st1, st2], axis=1)


def _p2_kernel(x_ref, y1_ref, w1_ref, ss_ref, o_ref, *, G, Cin, Cout):
    xb = x_ref[0]
    ss = ss_ref[...]
    sc = ss[0][:, 0:1]
    sh = ss[1][:, 0:1]
    b1c = ss[2][:, 0:1]
    for i in range(G):
        y2 = jnp.dot(w1_ref[...], xb[i * Cin:(i + 1) * Cin, :],
                     preferred_element_type=jnp.float32)
        y1 = y1_ref[0, i * Cout:(i + 1) * Cout, :].astype(jnp.float32)
        y1n = jnp.maximum(y1 * sc + sh, 0.0)
        o_ref[0, i * Cout:(i + 1) * Cout, :] = jnp.maximum(
            y1n + y2 + b1c, 0.0).astype(jnp.bfloat16)


@jax.jit
def _forward(x_nchw, w3, b3, gamma, beta, w1, b1):
    N, Cin, H, W = x_nchw.shape
    Cout = w3.shape[-1]
    HW = H * W
    P = N * HW
    g = math.gcd(GIMG, N)
    ng = N // g
    pb = math.gcd(PB, HW)
    np_ = HW // pb

    cparams = pltpu.CompilerParams(
        dimension_semantics=("parallel",),
        vmem_limit_bytes=64 * 1024 * 1024,
    )

    xv = jnp.transpose(x_nchw, (1, 2, 3, 0)).reshape(Cin, HW, N)
    xv = xv.astype(jnp.float32)
    xt = pl.pallas_call(
        _tin_kernel,
        grid=(np_,),
        in_specs=[pl.BlockSpec((Cin, pb, N), lambda j: (0, j, 0))],
        out_specs=pl.BlockSpec((N, Cin, pb), lambda j: (0, 0, j)),
        out_shape=jax.ShapeDtypeStruct((N, Cin, HW), jnp.bfloat16),
        compiler_params=cparams,
        cost_estimate=pl.CostEstimate(
            flops=0, transcendentals=0,
            bytes_accessed=int(4 * Cin * HW * N + 2 * Cin * HW * N)),
    )(xv)
    x = xt.reshape(ng, g * Cin, HW)

    w9 = jnp.transpose(w3.astype(jnp.float32),
                       (0, 1, 3, 2)).reshape(9, Cout, Cin).astype(jnp.bfloat16)
    w1t = jnp.transpose(w1.astype(jnp.float32)).astype(jnp.bfloat16)
    b3b = jnp.broadcast_to(b3.reshape(Cout, 1).astype(jnp.float32),
                           (Cout, 128))

    flops1 = int(N * 9 * Cout * Cin * HW * 2 + N * 6 * Cout * HW)
    bytes1 = int(2 * N * Cin * HW + 2 * N * Cout * HW + 2 * 9 * Cout * Cin
                 + 4 * (Cout * 128 + ng * Cout * 2))
    y1, stats = pl.pallas_call(
        partial(_p1_kernel, G=g, W=W, Cin=Cin, Cout=Cout),
        grid=(ng,),
        in_specs=[
            pl.BlockSpec((1, g * Cin, HW), lambda n: (n, 0, 0)),
            pl.BlockSpec((9, Cout, Cin), lambda n: (0, 0, 0)),
            pl.BlockSpec((Cout, 128), lambda n: (0, 0)),
        ],
        out_specs=(
            pl.BlockSpec((1, g * Cout, HW), lambda n: (n, 0, 0)),
            pl.BlockSpec((1, Cout, 2), lambda n: (n, 0, 0)),
        ),
        out_shape=(
            jax.ShapeDtypeStruct((ng, g * Cout, HW), jnp.bfloat16),
            jax.ShapeDtypeStruct((ng, Cout, 2), jnp.float32),
        ),
        compiler_params=cparams,
        cost_estimate=pl.CostEstimate(flops=flops1, transcendentals=0,
                                      bytes_accessed=bytes1),
    )(x, w9, b3b)

    s = stats.sum(axis=0)
    mean = s[:, 0] / P
    var = s[:, 1] / P - mean * mean
    scale = gamma.reshape(Cout) * lax.rsqrt(var + EPS)
    shift = beta.reshape(Cout) - mean * scale
    ssb = jnp.broadcast_to(
        jnp.stack([scale, shift, b1.reshape(Cout).astype(jnp.float32)]
                  )[:, :, None], (3, Cout, 128))

    flops2 = int(N * Cout * Cin * HW * 2 + N * 6 * Cout * HW)
    bytes2 = int(2 * N * Cin * HW + 2 * 2 * N * Cout * HW + 2 * Cout * Cin
                 + 4 * 3 * Cout * 128)
    ot = pl.pallas_call(
        partial(_p2_kernel, G=g, Cin=Cin, Cout=Cout),
        grid=(ng,),
        in_specs=[
            pl.BlockSpec((1, g * Cin, HW), lambda n: (n, 0, 0)),
            pl.BlockSpec((1, g * Cout, HW), lambda n: (n, 0, 0)),
            pl.BlockSpec((Cout, Cin), lambda n: (0, 0)),
            pl.BlockSpec((3, Cout, 128), lambda n: (0, 0, 0)),
        ],
        out_specs=pl.BlockSpec((1, g * Cout, HW), lambda n: (n, 0, 0)),
        out_shape=jax.ShapeDtypeStruct((ng, g * Cout, HW), jnp.bfloat16),
        compiler_params=cparams,
        cost_estimate=pl.CostEstimate(flops=flops2, transcendentals=0,
                                      bytes_accessed=bytes2),
    )(x, y1, w1t, ssb)

    oc = pl.pallas_call(
        _tout_kernel,
        grid=(np_,),
        in_specs=[pl.BlockSpec((N, Cout, pb), lambda j: (0, 0, j))],
        out_specs=pl.BlockSpec((Cout, pb, N), lambda j: (0, j, 0)),
        out_shape=jax.ShapeDtypeStruct((Cout, HW, N), jnp.float32),
        compiler_params=cparams,
        cost_estimate=pl.CostEstimate(
            flops=0, transcendentals=0,
            bytes_accessed=int(2 * Cout * HW * N + 4 * Cout * HW * N)),
    )(ot.reshape(N, Cout, HW))

    return jnp.transpose(oc.reshape(Cout, H, W, N), (3, 0, 1, 2))


def kernel(x_nchw, w3, b3, gamma, beta, w1, b1):
    return _forward(x_nchw, w3, b3, gamma, beta, w1, b1)


# speedup vs baseline: 2.8850x; 1.0769x over previous
import math
from functools import partial

import jax
import jax.numpy as jnp
from jax import lax
from jax.experimental import pallas as pl
from jax.experimental.pallas import tpu as pltpu

EPS = 1e-5
GIMG = 8
PB = 128


def _shift_lanes(x, s, zcol):
    if s == 0:
        return x
    if s > 0:
        return jnp.concatenate([x[:, s:], zcol[:, :s]], axis=1)
    return jnp.concatenate([zcol[:, :(-s)], x[:, :s]], axis=1)


def _tin_kernel(x_ref, o_ref):
    o_ref[...] = jnp.transpose(x_ref[...], (2, 0, 1)).astype(jnp.bfloat16)


def _p1_kernel(x_ref, w_ref, b3_ref, y1_ref, st_ref, *, G, W, Cin, Cout):
    xb = x_ref[0]
    rows, hw = xb.shape
    zcol = jnp.zeros((rows, 33), jnp.bfloat16)
    lane = lax.broadcasted_iota(jnp.int32, (1, hw), 1) % W
    zero = jnp.zeros((), jnp.bfloat16)
    shifted = []
    for ky in range(3):
        for kx in range(3):
            s = W * (ky - 1) + (kx - 1)
            t = _shift_lanes(xb, s, zcol)
            if kx == 0:
                t = jnp.where(lane == 0, zero, t)
            elif kx == 2:
                t = jnp.where(lane == W - 1, zero, t)
            shifted.append(t)
    b3c = b3_ref[:, 0:1]
    for i in range(G):
        r0 = i * Cin
        acc = jnp.dot(w_ref[0], shifted[0][r0:r0 + Cin, :],
                      preferred_element_type=jnp.float32)
        for k in range(1, 9):
            acc = acc + jnp.dot(w_ref[k], shifted[k][r0:r0 + Cin, :],
                                preferred_element_type=jnp.float32)
        y = acc + b3c
        y1_ref[0, i * Cout:(i + 1) * Cout, :] = y.astype(jnp.bfloat16)
        s1 = jnp.sum(y, axis=1, keepdims=True)
        s2 = jnp.sum(y * y, axis=1, keepdims=True)
        if i == 0:
            st1, st2 = s1, s2
        else:
            st1, st2 = st1 + s1, st2 + s2
    st_ref[0] = jnp.concatenate([st1, st2], axis=1)


def _p2_kernel(xv_ref, y1_ref, w1_ref, ss_ref, o_ref, *, Cin, Cout, N):
    pb = xv_ref.shape[1]
    xb = xv_ref[...].reshape(Cin, pb * N).astype(jnp.bfloat16)
    y2 = jnp.dot(w1_ref[...], xb,
                 preferred_element_type=jnp.float32).reshape(Cout, pb, N)
    y1c = jnp.transpose(y1_ref[...], (1, 2, 0)).astype(jnp.float32)
    ss = ss_ref[...]
    sc = ss[0][:, 0:1, None]
    sh = ss[1][:, 0:1, None]
    b1c = ss[2][:, 0:1, None]
    y1n = jnp.maximum(y1c * sc + sh, 0.0)
    o_ref[...] = jnp.maximum(y1n + y2 + b1c, 0.0)


@jax.jit
def _forward(x_nchw, w3, b3, gamma, beta, w1, b1):
    N, Cin, H, W = x_nchw.shape
    Cout = w3.shape[-1]
    HW = H * W
    P = N * HW
    g = math.gcd(GIMG, N)
    ng = N // g
    pb = math.gcd(PB, HW)
    np_ = HW // pb

    cparams = pltpu.CompilerParams(
        dimension_semantics=("parallel",),
        vmem_limit_bytes=64 * 1024 * 1024,
    )

    xv = jnp.transpose(x_nchw, (1, 2, 3, 0)).reshape(Cin, HW, N)
    xv = xv.astype(jnp.float32)
    xt = pl.pallas_call(
        _tin_kernel,
        grid=(np_,),
        in_specs=[pl.BlockSpec((Cin, pb, N), lambda j: (0, j, 0))],
        out_specs=pl.BlockSpec((N, Cin, pb), lambda j: (0, 0, j)),
        out_shape=jax.ShapeDtypeStruct((N, Cin, HW), jnp.bfloat16),
        compiler_params=cparams,
        cost_estimate=pl.CostEstimate(
            flops=0, transcendentals=0,
            bytes_accessed=int(4 * Cin * HW * N + 2 * Cin * HW * N)),
    )(xv)
    x = xt.reshape(ng, g * Cin, HW)

    w9 = jnp.transpose(w3.astype(jnp.float32),
                       (0, 1, 3, 2)).reshape(9, Cout, Cin).astype(jnp.bfloat16)
    w1t = jnp.transpose(w1.astype(jnp.float32)).astype(jnp.bfloat16)
    b3b = jnp.broadcast_to(b3.reshape(Cout, 1).astype(jnp.float32),
                           (Cout, 128))

    flops1 = int(N * 9 * Cout * Cin * HW * 2 + N * 6 * Cout * HW)
    bytes1 = int(2 * N * Cin * HW + 2 * N * Cout * HW + 2 * 9 * Cout * Cin
                 + 4 * (Cout * 128 + ng * Cout * 2))
    y1, stats = pl.pallas_call(
        partial(_p1_kernel, G=g, W=W, Cin=Cin, Cout=Cout),
        grid=(ng,),
        in_specs=[
            pl.BlockSpec((1, g * Cin, HW), lambda n: (n, 0, 0)),
            pl.BlockSpec((9, Cout, Cin), lambda n: (0, 0, 0)),
            pl.BlockSpec((Cout, 128), lambda n: (0, 0)),
        ],
        out_specs=(
            pl.BlockSpec((1, g * Cout, HW), lambda n: (n, 0, 0)),
            pl.BlockSpec((1, Cout, 2), lambda n: (n, 0, 0)),
        ),
        out_shape=(
            jax.ShapeDtypeStruct((ng, g * Cout, HW), jnp.bfloat16),
            jax.ShapeDtypeStruct((ng, Cout, 2), jnp.float32),
        ),
        compiler_params=cparams,
        cost_estimate=pl.CostEstimate(flops=flops1, transcendentals=0,
                                      bytes_accessed=bytes1),
    )(x, w9, b3b)

    s = stats.sum(axis=0)
    mean = s[:, 0] / P
    var = s[:, 1] / P - mean * mean
    scale = gamma.reshape(Cout) * lax.rsqrt(var + EPS)
    shift = beta.reshape(Cout) - mean * scale
    ssb = jnp.broadcast_to(
        jnp.stack([scale, shift, b1.reshape(Cout).astype(jnp.float32)]
                  )[:, :, None], (3, Cout, 128))

    flops2 = int(N * Cout * Cin * HW * 2 + N * 6 * Cout * HW)
    bytes2 = int(4 * N * Cin * HW + 2 * N * Cout * HW + 2 * Cout * Cin
                 + 4 * 3 * Cout * 128 + 4 * N * Cout * HW)
    oc = pl.pallas_call(
        partial(_p2_kernel, Cin=Cin, Cout=Cout, N=N),
        grid=(np_,),
        in_specs=[
            pl.BlockSpec((Cin, pb, N), lambda j: (0, j, 0)),
            pl.BlockSpec((N, Cout, pb), lambda j: (0, 0, j)),
            pl.BlockSpec((Cout, Cin), lambda j: (0, 0)),
            pl.BlockSpec((3, Cout, 128), lambda j: (0, 0, 0)),
        ],
        out_specs=pl.BlockSpec((Cout, pb, N), lambda j: (0, j, 0)),
        out_shape=jax.ShapeDtypeStruct((Cout, HW, N), jnp.float32),
        compiler_params=cparams,
        cost_estimate=pl.CostEstimate(flops=flops2, transcendentals=0,
                                      bytes_accessed=bytes2),
    )(xv, y1.reshape(N, Cout, HW), w1t, ssb)

    return jnp.transpose(oc.reshape(Cout, H, W, N), (3, 0, 1, 2))


def kernel(x_nchw, w3, b3, gamma, beta, w1, b1):
    return _forward(x_nchw, w3, b3, gamma, beta, w1, b1)


# speedup vs baseline: 2.9640x; 1.0274x over previous
import math
from functools import partial

import jax
import jax.numpy as jnp
from jax import lax
from jax.experimental import pallas as pl
from jax.experimental.pallas import tpu as pltpu

EPS = 1e-5
GIMG = 16
PB = 128


def _shift_lanes(x, s, zcol):
    if s == 0:
        return x
    if s > 0:
        return jnp.concatenate([x[:, s:], zcol[:, :s]], axis=1)
    return jnp.concatenate([zcol[:, :(-s)], x[:, :s]], axis=1)


def _tin_kernel(x_ref, o_ref):
    o_ref[...] = jnp.transpose(x_ref[...], (2, 0, 1)).astype(jnp.bfloat16)


def _p1_kernel(x_ref, w_ref, b3_ref, y1_ref, st_ref, *, G, W, Cin, Cout):
    xb = x_ref[0]
    rows, hw = xb.shape
    zcol = jnp.zeros((rows, 33), jnp.bfloat16)
    lane = lax.broadcasted_iota(jnp.int32, (1, hw), 1) % W
    zero = jnp.zeros((), jnp.bfloat16)
    shifted = []
    for ky in range(3):
        for kx in range(3):
            s = W * (ky - 1) + (kx - 1)
            t = _shift_lanes(xb, s, zcol)
            if kx == 0:
                t = jnp.where(lane == 0, zero, t)
            elif kx == 2:
                t = jnp.where(lane == W - 1, zero, t)
            shifted.append(t)
    b3c = b3_ref[:, 0:1]
    for i in range(G):
        r0 = i * Cin
        acc = jnp.dot(w_ref[0], shifted[0][r0:r0 + Cin, :],
                      preferred_element_type=jnp.float32)
        for k in range(1, 9):
            acc = acc + jnp.dot(w_ref[k], shifted[k][r0:r0 + Cin, :],
                                preferred_element_type=jnp.float32)
        y = acc + b3c
        y1_ref[0, i * Cout:(i + 1) * Cout, :] = y.astype(jnp.bfloat16)
        s1 = jnp.sum(y, axis=1, keepdims=True)
        s2 = jnp.sum(y * y, axis=1, keepdims=True)
        if i == 0:
            st1, st2 = s1, s2
        else:
            st1, st2 = st1 + s1, st2 + s2
    st_ref[0] = jnp.concatenate([st1, st2], axis=1)


def _p2_kernel(xv_ref, y1_ref, w1_ref, ss_ref, o_ref, *, Cin, Cout, N):
    pb = xv_ref.shape[1]
    xb = xv_ref[...].reshape(Cin, pb * N).astype(jnp.bfloat16)
    y2 = jnp.dot(w1_ref[...], xb,
                 preferred_element_type=jnp.float32).reshape(Cout, pb, N)
    y1c = jnp.transpose(y1_ref[...], (1, 2, 0)).astype(jnp.float32)
    ss = ss_ref[...]
    sc = ss[0][:, 0:1, None]
    sh = ss[1][:, 0:1, None]
    b1c = ss[2][:, 0:1, None]
    y1n = jnp.maximum(y1c * sc + sh, 0.0)
    o_ref[...] = jnp.maximum(y1n + y2 + b1c, 0.0)


@jax.jit
def _forward(x_nchw, w3, b3, gamma, beta, w1, b1):
    N, Cin, H, W = x_nchw.shape
    Cout = w3.shape[-1]
    HW = H * W
    P = N * HW
    g = math.gcd(GIMG, N)
    ng = N // g
    pb = math.gcd(PB, HW)
    np_ = HW // pb

    cparams = pltpu.CompilerParams(
        dimension_semantics=("parallel",),
        vmem_limit_bytes=64 * 1024 * 1024,
    )

    xv = jnp.transpose(x_nchw, (1, 2, 3, 0)).reshape(Cin, HW, N)
    xv = xv.astype(jnp.float32)
    xt = pl.pallas_call(
        _tin_kernel,
        grid=(np_,),
        in_specs=[pl.BlockSpec((Cin, pb, N), lambda j: (0, j, 0))],
        out_specs=pl.BlockSpec((N, Cin, pb), lambda j: (0, 0, j)),
        out_shape=jax.ShapeDtypeStruct((N, Cin, HW), jnp.bfloat16),
        compiler_params=cparams,
        cost_estimate=pl.CostEstimate(
            flops=0, transcendentals=0,
            bytes_accessed=int(4 * Cin * HW * N + 2 * Cin * HW * N)),
    )(xv)
    x = xt.reshape(ng, g * Cin, HW)

    w9 = jnp.transpose(w3.astype(jnp.float32),
                       (0, 1, 3, 2)).reshape(9, Cout, Cin).astype(jnp.bfloat16)
    w1t = jnp.transpose(w1.astype(jnp.float32)).astype(jnp.bfloat16)
    b3b = jnp.broadcast_to(b3.reshape(Cout, 1).astype(jnp.float32),
                           (Cout, 128))

    flops1 = int(N * 9 * Cout * Cin * HW * 2 + N * 6 * Cout * HW)
    bytes1 = int(2 * N * Cin * HW + 2 * N * Cout * HW + 2 * 9 * Cout * Cin
                 + 4 * (Cout * 128 + ng * Cout * 2))
    y1, stats = pl.pallas_call(
        partial(_p1_kernel, G=g, W=W, Cin=Cin, Cout=Cout),
        grid=(ng,),
        in_specs=[
            pl.BlockSpec((1, g * Cin, HW), lambda n: (n, 0, 0)),
            pl.BlockSpec((9, Cout, Cin), lambda n: (0, 0, 0)),
            pl.BlockSpec((Cout, 128), lambda n: (0, 0)),
        ],
        out_specs=(
            pl.BlockSpec((1, g * Cout, HW), lambda n: (n, 0, 0)),
            pl.BlockSpec((1, Cout, 2), lambda n: (n, 0, 0)),
        ),
        out_shape=(
            jax.ShapeDtypeStruct((ng, g * Cout, HW), jnp.bfloat16),
            jax.ShapeDtypeStruct((ng, Cout, 2), jnp.float32),
        ),
        compiler_params=cparams,
        cost_estimate=pl.CostEstimate(flops=flops1, transcendentals=0,
                                      bytes_accessed=bytes1),
    )(x, w9, b3b)

    s = stats.sum(axis=0)
    mean = s[:, 0] / P
    var = s[:, 1] / P - mean * mean
    scale = gamma.reshape(Cout) * lax.rsqrt(var + EPS)
    shift = beta.reshape(Cout) - mean * scale
    ssb = jnp.broadcast_to(
        jnp.stack([scale, shift, b1.reshape(Cout).astype(jnp.float32)]
                  )[:, :, None], (3, Cout, 128))

    flops2 = int(N * Cout * Cin * HW * 2 + N * 6 * Cout * HW)
    bytes2 = int(4 * N * Cin * HW + 2 * N * Cout * HW + 2 * Cout * Cin
                 + 4 * 3 * Cout * 128 + 4 * N * Cout * HW)
    oc = pl.pallas_call(
        partial(_p2_kernel, Cin=Cin, Cout=Cout, N=N),
        grid=(np_,),
        in_specs=[
            pl.BlockSpec((Cin, pb, N), lambda j: (0, j, 0)),
            pl.BlockSpec((N, Cout, pb), lambda j: (0, 0, j)),
            pl.BlockSpec((Cout, Cin), lambda j: (0, 0)),
            pl.BlockSpec((3, Cout, 128), lambda j: (0, 0, 0)),
        ],
        out_specs=pl.BlockSpec((Cout, pb, N), lambda j: (0, j, 0)),
        out_shape=jax.ShapeDtypeStruct((Cout, HW, N), jnp.float32),
        compiler_params=cparams,
        cost_estimate=pl.CostEstimate(flops=flops2, transcendentals=0,
                                      bytes_accessed=bytes2),
    )(xv, y1.reshape(N, Cout, HW), w1t, ssb)

    return jnp.transpose(oc.reshape(Cout, H, W, N), (3, 0, 1, 2))


def kernel(x_nchw, w3, b3, gamma, beta, w1, b1):
    return _forward(x_nchw, w3, b3, gamma, beta, w1, b1)


# speedup vs baseline: 3.0112x; 1.0159x over previous
import math
from functools import partial

import jax
import jax.numpy as jnp
from jax import lax
from jax.experimental import pallas as pl
from jax.experimental.pallas import tpu as pltpu

EPS = 1e-5
GIMG = 32
PB = 128


def _shift_lanes(x, s, zcol):
    if s == 0:
        return x
    if s > 0:
        return jnp.concatenate([x[:, s:], zcol[:, :s]], axis=1)
    return jnp.concatenate([zcol[:, :(-s)], x[:, :s]], axis=1)


def _tin_kernel(x_ref, o_ref):
    o_ref[...] = jnp.transpose(x_ref[...], (2, 0, 1)).astype(jnp.bfloat16)


def _p1_kernel(x_ref, w_ref, b3_ref, y1_ref, st_ref, *, G, W, Cin, Cout):
    xb = x_ref[0]
    rows, hw = xb.shape
    zcol = jnp.zeros((rows, 33), jnp.bfloat16)
    lane = lax.broadcasted_iota(jnp.int32, (1, hw), 1) % W
    zero = jnp.zeros((), jnp.bfloat16)
    shifted = []
    for ky in range(3):
        for kx in range(3):
            s = W * (ky - 1) + (kx - 1)
            t = _shift_lanes(xb, s, zcol)
            if kx == 0:
                t = jnp.where(lane == 0, zero, t)
            elif kx == 2:
                t = jnp.where(lane == W - 1, zero, t)
            shifted.append(t)
    b3c = b3_ref[:, 0:1]
    for i in range(G):
        r0 = i * Cin
        acc = jnp.dot(w_ref[0], shifted[0][r0:r0 + Cin, :],
                      preferred_element_type=jnp.float32)
        for k in range(1, 9):
            acc = acc + jnp.dot(w_ref[k], shifted[k][r0:r0 + Cin, :],
                                preferred_element_type=jnp.float32)
        y = acc + b3c
        y1_ref[0, i * Cout:(i + 1) * Cout, :] = y.astype(jnp.bfloat16)
        s1 = jnp.sum(y, axis=1, keepdims=True)
        s2 = jnp.sum(y * y, axis=1, keepdims=True)
        if i == 0:
            st1, st2 = s1, s2
        else:
            st1, st2 = st1 + s1, st2 + s2
    st_ref[0] = jnp.concatenate([st1, st2], axis=1)


def _p2_kernel(xv_ref, y1_ref, w1_ref, ss_ref, o_ref, *, Cin, Cout, N):
    pb = xv_ref.shape[1]
    xb = xv_ref[...].reshape(Cin, pb * N).astype(jnp.bfloat16)
    y2 = jnp.dot(w1_ref[...], xb,
                 preferred_element_type=jnp.float32).reshape(Cout, pb, N)
    y1c = jnp.transpose(y1_ref[...], (1, 2, 0)).astype(jnp.float32)
    ss = ss_ref[...]
    sc = ss[0][:, 0:1, None]
    sh = ss[1][:, 0:1, None]
    b1c = ss[2][:, 0:1, None]
    y1n = jnp.maximum(y1c * sc + sh, 0.0)
    o_ref[...] = jnp.maximum(y1n + y2 + b1c, 0.0)


@jax.jit
def _forward(x_nchw, w3, b3, gamma, beta, w1, b1):
    N, Cin, H, W = x_nchw.shape
    Cout = w3.shape[-1]
    HW = H * W
    P = N * HW
    g = math.gcd(GIMG, N)
    ng = N // g
    pb = math.gcd(PB, HW)
    np_ = HW // pb

    cparams = pltpu.CompilerParams(
        dimension_semantics=("parallel",),
        vmem_limit_bytes=64 * 1024 * 1024,
    )

    xv = jnp.transpose(x_nchw, (1, 2, 3, 0)).reshape(Cin, HW, N)
    xv = xv.astype(jnp.float32)
    xt = pl.pallas_call(
        _tin_kernel,
        grid=(np_,),
        in_specs=[pl.BlockSpec((Cin, pb, N), lambda j: (0, j, 0))],
        out_specs=pl.BlockSpec((N, Cin, pb), lambda j: (0, 0, j)),
        out_shape=jax.ShapeDtypeStruct((N, Cin, HW), jnp.bfloat16),
        compiler_params=cparams,
        cost_estimate=pl.CostEstimate(
            flops=0, transcendentals=0,
            bytes_accessed=int(4 * Cin * HW * N + 2 * Cin * HW * N)),
    )(xv)
    x = xt.reshape(ng, g * Cin, HW)

    w9 = jnp.transpose(w3.astype(jnp.float32),
                       (0, 1, 3, 2)).reshape(9, Cout, Cin).astype(jnp.bfloat16)
    w1t = jnp.transpose(w1.astype(jnp.float32)).astype(jnp.bfloat16)
    b3b = jnp.broadcast_to(b3.reshape(Cout, 1).astype(jnp.float32),
                           (Cout, 128))

    flops1 = int(N * 9 * Cout * Cin * HW * 2 + N * 6 * Cout * HW)
    bytes1 = int(2 * N * Cin * HW + 2 * N * Cout * HW + 2 * 9 * Cout * Cin
                 + 4 * (Cout * 128 + ng * Cout * 2))
    y1, stats = pl.pallas_call(
        partial(_p1_kernel, G=g, W=W, Cin=Cin, Cout=Cout),
        grid=(ng,),
        in_specs=[
            pl.BlockSpec((1, g * Cin, HW), lambda n: (n, 0, 0)),
            pl.BlockSpec((9, Cout, Cin), lambda n: (0, 0, 0)),
            pl.BlockSpec((Cout, 128), lambda n: (0, 0)),
        ],
        out_specs=(
            pl.BlockSpec((1, g * Cout, HW), lambda n: (n, 0, 0)),
            pl.BlockSpec((1, Cout, 2), lambda n: (n, 0, 0)),
        ),
        out_shape=(
            jax.ShapeDtypeStruct((ng, g * Cout, HW), jnp.bfloat16),
            jax.ShapeDtypeStruct((ng, Cout, 2), jnp.float32),
        ),
        compiler_params=cparams,
        cost_estimate=pl.CostEstimate(flops=flops1, transcendentals=0,
                                      bytes_accessed=bytes1),
    )(x, w9, b3b)

    s = stats.sum(axis=0)
    mean = s[:, 0] / P
    var = s[:, 1] / P - mean * mean
    scale = gamma.reshape(Cout) * lax.rsqrt(var + EPS)
    shift = beta.reshape(Cout) - mean * scale
    ssb = jnp.broadcast_to(
        jnp.stack([scale, shift, b1.reshape(Cout).astype(jnp.float32)]
                  )[:, :, None], (3, Cout, 128))

    flops2 = int(N * Cout * Cin * HW * 2 + N * 6 * Cout * HW)
    bytes2 = int(4 * N * Cin * HW + 2 * N * Cout * HW + 2 * Cout * Cin
                 + 4 * 3 * Cout * 128 + 4 * N * Cout * HW)
    oc = pl.pallas_call(
        partial(_p2_kernel, Cin=Cin, Cout=Cout, N=N),
        grid=(np_,),
        in_specs=[
            pl.BlockSpec((Cin, pb, N), lambda j: (0, j, 0)),
            pl.BlockSpec((N, Cout, pb), lambda j: (0, 0, j)),
            pl.BlockSpec((Cout, Cin), lambda j: (0, 0)),
            pl.BlockSpec((3, Cout, 128), lambda j: (0, 0, 0)),
        ],
        out_specs=pl.BlockSpec((Cout, pb, N), lambda j: (0, j, 0)),
        out_shape=jax.ShapeDtypeStruct((Cout, HW, N), jnp.float32),
        compiler_params=cparams,
        cost_estimate=pl.CostEstimate(flops=flops2, transcendentals=0,
                                      bytes_accessed=bytes2),
    )(xv, y1.reshape(N, Cout, HW), w1t, ssb)

    return jnp.transpose(oc.reshape(Cout, H, W, N), (3, 0, 1, 2))


def kernel(x_nchw, w3, b3, gamma, beta, w1, b1):
    return _forward(x_nchw, w3, b3, gamma, beta, w1, b1)


# speedup vs baseline: 3.0858x; 1.0248x over previous
import math
from functools import partial

import jax
import jax.numpy as jnp
from jax import lax
from jax.experimental import pallas as pl
from jax.experimental.pallas import tpu as pltpu

EPS = 1e-5
GIMG = 32
PB = 128


def _shift_lanes(x, s, zcol):
    if s == 0:
        return x
    if s > 0:
        return jnp.concatenate([x[:, s:], zcol[:, :s]], axis=1)
    return jnp.concatenate([zcol[:, :(-s)], x[:, :s]], axis=1)


def _tin_kernel(x_ref, o_ref):
    o_ref[...] = jnp.transpose(x_ref[...], (2, 0, 1)).astype(jnp.bfloat16)


def _p1_kernel(x_ref, w_ref, b3_ref, y1_ref, st_ref, *, G, W, Cin, Cout):
    xb = x_ref[0]
    rows, hw = xb.shape
    zcol = jnp.zeros((rows, 33), jnp.bfloat16)
    lane = lax.broadcasted_iota(jnp.int32, (1, hw), 1) % W
    zero = jnp.zeros((), jnp.bfloat16)
    shifted = []
    for ky in range(3):
        for kx in range(3):
            s = W * (ky - 1) + (kx - 1)
            t = _shift_lanes(xb, s, zcol)
            if kx == 0:
                t = jnp.where(lane == 0, zero, t)
            elif kx == 2:
                t = jnp.where(lane == W - 1, zero, t)
            shifted.append(t)
    b3c = b3_ref[:, 0:1]
    for i in range(G):
        r0 = i * Cin
        acc = jnp.dot(w_ref[0], shifted[0][r0:r0 + Cin, :],
                      preferred_element_type=jnp.float32)
        for k in range(1, 9):
            acc = acc + jnp.dot(w_ref[k], shifted[k][r0:r0 + Cin, :],
                                preferred_element_type=jnp.float32)
        y = acc + b3c
        y1_ref[0, i * Cout:(i + 1) * Cout, :] = y.astype(jnp.bfloat16)
        s1 = jnp.sum(y, axis=1, keepdims=True)
        s2 = jnp.sum(y * y, axis=1, keepdims=True)
        if i == 0:
            st1, st2 = s1, s2
        else:
            st1, st2 = st1 + s1, st2 + s2
    st_ref[0] = jnp.concatenate([st1, st2], axis=1)


def _p2_kernel(xv_ref, y1_ref, w1_ref, ss_ref, o_ref, *, Cin, Cout, N):
    pb = xv_ref.shape[1]
    xb = xv_ref[...].reshape(Cin, pb * N).astype(jnp.bfloat16)
    y2 = jnp.dot(w1_ref[...], xb,
                 preferred_element_type=jnp.float32).reshape(Cout, pb, N)
    y1c = jnp.transpose(y1_ref[...], (1, 2, 0)).astype(jnp.float32)
    ss = ss_ref[...]
    sc = ss[0][:, 0:1, None]
    sh = ss[1][:, 0:1, None]
    b1c = ss[2][:, 0:1, None]
    y1n = jnp.maximum(y1c * sc + sh, 0.0)
    o_ref[...] = jnp.maximum(y1n + y2 + b1c, 0.0)


@jax.jit
def _forward(x_nchw, w3, b3, gamma, beta, w1, b1):
    N, Cin, H, W = x_nchw.shape
    Cout = w3.shape[-1]
    HW = H * W
    P = N * HW
    g = math.gcd(GIMG, N)
    ng = N // g
    pb = math.gcd(PB, HW)
    np_ = HW // pb

    cparams = pltpu.CompilerParams(
        dimension_semantics=("parallel",),
        vmem_limit_bytes=64 * 1024 * 1024,
    )

    xv = jnp.transpose(x_nchw, (1, 2, 3, 0)).reshape(Cin, HW, N)
    xv = xv.astype(jnp.float32)
    xt = pl.pallas_call(
        _tin_kernel,
        grid=(np_,),
        in_specs=[pl.BlockSpec((Cin, pb, N), lambda j: (0, j, 0))],
        out_specs=pl.BlockSpec((N, Cin, pb), lambda j: (0, 0, j)),
        out_shape=jax.ShapeDtypeStruct((N, Cin, HW), jnp.bfloat16),
        compiler_params=cparams,
        cost_estimate=pl.CostEstimate(
            flops=0, transcendentals=0,
            bytes_accessed=int(4 * Cin * HW * N + 2 * Cin * HW * N)),
    )(xv)
    x = xt.reshape(ng, g * Cin, HW)

    w9 = jnp.transpose(w3.astype(jnp.float32),
                       (0, 1, 3, 2)).reshape(9, Cout, Cin).astype(jnp.bfloat16)
    w1t = jnp.transpose(w1.astype(jnp.float32)).astype(jnp.bfloat16)
    b3b = jnp.broadcast_to(b3.reshape(Cout, 1).astype(jnp.float32),
                           (Cout, 128))

    flops1 = int(N * 9 * Cout * Cin * HW * 2 + N * 6 * Cout * HW)
    bytes1 = int(2 * N * Cin * HW + 2 * N * Cout * HW + 2 * 9 * Cout * Cin
                 + 4 * (Cout * 128 + ng * Cout * 2))
    y1, stats = pl.pallas_call(
        partial(_p1_kernel, G=g, W=W, Cin=Cin, Cout=Cout),
        grid=(ng,),
        in_specs=[
            pl.BlockSpec((1, g * Cin, HW), lambda n: (n, 0, 0)),
            pl.BlockSpec((9, Cout, Cin), lambda n: (0, 0, 0)),
            pl.BlockSpec((Cout, 128), lambda n: (0, 0)),
        ],
        out_specs=(
            pl.BlockSpec((1, g * Cout, HW), lambda n: (n, 0, 0)),
            pl.BlockSpec((1, Cout, 2), lambda n: (n, 0, 0)),
        ),
        out_shape=(
            jax.ShapeDtypeStruct((ng, g * Cout, HW), jnp.bfloat16),
            jax.ShapeDtypeStruct((ng, Cout, 2), jnp.float32),
        ),
        compiler_params=cparams,
        cost_estimate=pl.CostEstimate(flops=flops1, transcendentals=0,
                                      bytes_accessed=bytes1),
    )(x, w9, b3b)

    s = stats.sum(axis=0)
    mean = s[:, 0] / P
    var = s[:, 1] / P - mean * mean
    scale = gamma.reshape(Cout) * lax.rsqrt(var + EPS)
    shift = beta.reshape(Cout) - mean * scale
    ssb = jnp.broadcast_to(
        jnp.stack([scale, shift, b1.reshape(Cout).astype(jnp.float32)]
                  )[:, :, None], (3, Cout, 128))

    flops2 = int(N * Cout * Cin * HW * 2 + N * 6 * Cout * HW)
    bytes2 = int(4 * N * Cin * HW + 2 * N * Cout * HW + 2 * Cout * Cin
                 + 4 * 3 * Cout * 128 + 4 * N * Cout * HW)
    nb = min(128, N)
    nnb = N // nb
    cparams2 = pltpu.CompilerParams(
        dimension_semantics=("parallel", "parallel"),
        vmem_limit_bytes=64 * 1024 * 1024,
    )
    oc = pl.pallas_call(
        partial(_p2_kernel, Cin=Cin, Cout=Cout, N=nb),
        grid=(nnb, np_),
        in_specs=[
            pl.BlockSpec((Cin, pb, nb), lambda b, j: (0, j, b)),
            pl.BlockSpec((nb, Cout, pb), lambda b, j: (b, 0, j)),
            pl.BlockSpec((Cout, Cin), lambda b, j: (0, 0)),
            pl.BlockSpec((3, Cout, 128), lambda b, j: (0, 0, 0)),
        ],
        out_specs=pl.BlockSpec((Cout, pb, nb), lambda b, j: (0, j, b)),
        out_shape=jax.ShapeDtypeStruct((Cout, HW, N), jnp.float32),
        compiler_params=cparams2,
        cost_estimate=pl.CostEstimate(flops=flops2, transcendentals=0,
                                      bytes_accessed=bytes2),
    )(xv, y1.reshape(N, Cout, HW), w1t, ssb)

    return jnp.transpose(oc.reshape(Cout, H, W, N), (3, 0, 1, 2))


def kernel(x_nchw, w3, b3, gamma, beta, w1, b1):
    return _forward(x_nchw, w3, b3, gamma, beta, w1, b1)
